# Initial kernel scaffold; baseline (speedup 1.0000x reference)
import jax
import jax.numpy as jnp
from jax.experimental import pallas as pl


def kernel(x, grid1_table, grid0_table):
    raise NotImplementedError("write your pallas kernel here")


# speedup vs baseline: 69.0295x; 69.0295x over previous
import functools

import jax
import jax.numpy as jnp
from jax import lax
from jax.experimental import pallas as pl
from jax.experimental.pallas import tpu as pltpu
from jax.experimental.pallas import tpu_sc as plsc

RES_UP = 688
RES_DN = 20
N_PTS = 4194304

NC = 2
NS = 16
NW = NC * NS
L = 16

PER_TILE = N_PTS // NW
CHUNK = 2048
NCHUNK = PER_TILE // CHUNK
VPC = CHUNK // L
GB = 128
NG = CHUNK // GB

SG0_WORDS = RES_DN * RES_DN * RES_DN * 3
SG0_PAD = 24064


def _sigmoid_tc_body(x_ref, o_ref):
    o_ref[...] = jax.nn.sigmoid(x_ref[...])


def _sigmoid_tc(x2d):
    return pl.pallas_call(
        _sigmoid_tc_body,
        out_shape=jax.ShapeDtypeStruct(x2d.shape, x2d.dtype),
    )(x2d)


def _lerp(a, b, t):
    return a + t * (b - a)


def _sc_body(xf, t3, sg0, outf, xbuf, idxbuf, fubuf, fvbuf, rowsbuf, outbuf,
             sg0buf, sem):
    wid = lax.axis_index("s") * NC + lax.axis_index("c")
    base_pt = wid * PER_TILE
    lane = lax.iota(jnp.int32, L)

    pltpu.sync_copy(sg0, sg0buf)

    @pl.loop(0, NCHUNK)
    def _chunk(ch):
        pt0 = base_pt + ch * CHUNK
        pltpu.sync_copy(xf.at[pl.ds(pt0 * 2, CHUNK * 2)], xbuf)

        @pl.loop(0, VPC)
        def _p1(v):
            i2 = lane * 2 + v * (2 * L)
            x0 = plsc.load_gather(xbuf, [i2])
            x1 = plsc.load_gather(xbuf, [i2 + 1])
            u = jnp.minimum(jnp.maximum(x0, 0.0), 1.0) * (RES_UP - 1)
            w = jnp.minimum(jnp.maximum(x1, 0.0), 1.0) * (RES_UP - 1)
            ui = jnp.minimum(u.astype(jnp.int32), RES_UP - 2)
            vi = jnp.minimum(w.astype(jnp.int32), RES_UP - 2)
            sl = pl.ds(v * L, L)
            idxbuf[sl] = ui * RES_UP + vi
            fubuf[sl] = u - ui.astype(jnp.float32)
            fvbuf[sl] = w - vi.astype(jnp.float32)

        copies = []
        for j in range(NG):
            copies.append(pltpu.async_copy(
                t3.at[idxbuf.at[pl.ds(j * GB, GB)]],
                rowsbuf.at[pl.ds(j * GB, GB)], sem))
        for c in copies:
            c.wait()

        @pl.loop(0, VPC)
        def _p3(v):
            sl = pl.ds(v * L, L)
            fu = fubuf[sl]
            fv = fvbuf[sl]
            pidx = lane + v * L
            g1 = [plsc.load_gather(rowsbuf, [pidx, jnp.full((L,), c, jnp.int32)])
                  for c in range(12)]
            key = []
            for c in range(3):
                a = _lerp(g1[c], g1[3 + c], fv)
                b = _lerp(g1[6 + c], g1[9 + c], fv)
                key.append(_lerp(a, b, fu))

            t0 = key[0] * (RES_DN - 1)
            t1 = key[1] * (RES_DN - 1)
            t2 = key[2] * (RES_DN - 1)
            i0 = jnp.minimum(t0.astype(jnp.int32), RES_DN - 2)
            i1 = jnp.minimum(t1.astype(jnp.int32), RES_DN - 2)
            i2 = jnp.minimum(t2.astype(jnp.int32), RES_DN - 2)
            f0 = t0 - i0.astype(jnp.float32)
            f1 = t1 - i1.astype(jnp.float32)
            f2 = t2 - i2.astype(jnp.float32)
            b0 = i0 * (RES_DN * RES_DN * 3) + i1 * (RES_DN * 3) + i2 * 3

            for c in range(3):
                g = {}
                for du in (0, 1):
                    for dv in (0, 1):
                        for dw in (0, 1):
                            off = (du * (RES_DN * RES_DN * 3)
                                   + dv * (RES_DN * 3) + dw * 3 + c)
                            g[(du, dv, dw)] = plsc.load_gather(sg0buf, [b0 + off])
                c00 = _lerp(g[(0, 0, 0)], g[(0, 0, 1)], f2)
                c01 = _lerp(g[(0, 1, 0)], g[(0, 1, 1)], f2)
                c10 = _lerp(g[(1, 0, 0)], g[(1, 0, 1)], f2)
                c11 = _lerp(g[(1, 1, 0)], g[(1, 1, 1)], f2)
                c0 = _lerp(c00, c01, f1)
                c1 = _lerp(c10, c11, f1)
                res = _lerp(c0, c1, f0)
                plsc.store_scatter(outbuf, [lane * 3 + (v * (3 * L) + c)], res)

        pltpu.sync_copy(outbuf, outf.at[pl.ds(pt0 * 3, CHUNK * 3)])


@functools.partial(
    pl.kernel,
    out_type=jax.ShapeDtypeStruct((N_PTS * 3,), jnp.float32),
    mesh=plsc.VectorSubcoreMesh(core_axis_name="c", subcore_axis_name="s",
                                num_cores=NC, num_subcores=NS),
    compiler_params=pltpu.CompilerParams(needs_layout_passes=False,
                                         use_tc_tiling_on_sc=False),
    scratch_types=[
        pltpu.VMEM((CHUNK * 2,), jnp.float32),
        pltpu.VMEM((CHUNK,), jnp.int32),
        pltpu.VMEM((CHUNK,), jnp.float32),
        pltpu.VMEM((CHUNK,), jnp.float32),
        pltpu.VMEM((CHUNK, 16), jnp.float32),
        pltpu.VMEM((CHUNK * 3,), jnp.float32),
        pltpu.VMEM((SG0_PAD,), jnp.float32),
        pltpu.SemaphoreType.DMA,
    ],
)
def _sc_interp(xf, t3, sg0, outf, *rest):
    _sc_body(xf, t3, sg0, outf, *rest)


def kernel(x, grid1_table, grid0_table):
    sg1 = _sigmoid_tc(grid1_table.reshape(RES_UP * RES_UP * 3 // 128, 128))
    sg1 = sg1.reshape(RES_UP, RES_UP, 3)
    g0 = jnp.pad(grid0_table.reshape(-1), (0, SG0_PAD - SG0_WORDS))
    sg0 = _sigmoid_tc(g0.reshape(SG0_PAD // 128, 128)).reshape(-1)

    sg1p = jnp.pad(sg1, ((0, 1), (0, 1), (0, 0)))
    t3 = jnp.concatenate(
        [sg1p[:-1, :-1, :], sg1p[:-1, 1:, :], sg1p[1:, :-1, :], sg1p[1:, 1:, :],
         jnp.zeros((RES_UP, RES_UP, 4), jnp.float32)],
        axis=-1).reshape(RES_UP * RES_UP, 16)

    outf = _sc_interp(x.reshape(-1), t3, sg0)
    return outf.reshape(N_PTS, 3)
